# baseline (speedup 1.0000x reference)
import functools
import math

import jax
import jax.numpy as jnp
from jax.experimental import pallas as pl
from jax.experimental.pallas import tpu as pltpu

HPG = 4
CHUNK = 256


def _attn_kernel(x_ref, wqkv_ref, bqkv_ref, wo_ref, bo_ref, out_ref, ctx_ref,
                 *, n, dph, scale):
    g = pl.program_id(1)
    q_cols = HPG * dph

    xb = x_ref[0]
    qkv = jax.lax.dot_general(
        xb, wqkv_ref[0], (((1,), (0,)), ((), ())),
        preferred_element_type=jnp.float32) + bqkv_ref[0]

    log2e_scale = scale * 1.4426950408889634
    for h in range(HPG):
        q = (qkv[:, h * dph:(h + 1) * dph] * log2e_scale).astype(jnp.bfloat16)
        k = qkv[:, q_cols + h * dph:q_cols + (h + 1) * dph].astype(jnp.bfloat16)
        v = qkv[:, 2 * q_cols + h * dph:2 * q_cols + (h + 1) * dph]
        for c in range(n // CHUNK):
            qc = q[c * CHUNK:(c + 1) * CHUNK]
            s = jax.lax.dot_general(
                qc, k, (((1,), (1,)), ((), ())),
                preferred_element_type=jnp.float32)
            m = jnp.max(s, axis=1, keepdims=True)
            e = jnp.exp2(s - m)
            l = jnp.sum(e, axis=1)
            cc_t = jax.lax.dot_general(
                v, e, (((0,), (1,)), ((), ())),
                preferred_element_type=jnp.float32)
            cc_t = cc_t / l[None, :]
            ctx_ref[h * dph:(h + 1) * dph, c * CHUNK:(c + 1) * CHUNK] = (
                cc_t.astype(jnp.bfloat16))

    wo = wo_ref[0]
    for c in range(n // CHUNK):
        rows = slice(c * CHUNK, (c + 1) * CHUNK)
        contrib = jax.lax.dot_general(
            ctx_ref[:, rows], wo, (((0,), (0,)), ((), ())),
            preferred_element_type=jnp.float32)

        @pl.when(g == 0)
        def _():
            out_ref[0, rows, :] = contrib + bo_ref[...]

        @pl.when(g != 0)
        def _():
            out_ref[0, rows, :] = out_ref[0, rows, :] + contrib


def kernel(x, Wq, bq, Wk, bk, Wv, bv, Wo, bo):
    B, N, D = x.shape
    H, _, DPH = Wq.shape
    G = H // HPG
    q_cols = HPG * DPH

    def group_w(W):
        return W.reshape(G, HPG, D, DPH).transpose(0, 2, 1, 3).reshape(
            G, D, q_cols)

    Wqkv = jnp.concatenate([group_w(Wq), group_w(Wk), group_w(Wv)],
                           axis=2).astype(jnp.bfloat16)
    bqkv = jnp.concatenate(
        [bq.reshape(G, 1, q_cols), bk.reshape(G, 1, q_cols),
         bv.reshape(G, 1, q_cols)], axis=2)
    Wog = Wo.reshape(G, q_cols, D).astype(jnp.bfloat16)
    bo2 = bo.reshape(1, D)

    body = functools.partial(_attn_kernel, n=N, dph=DPH,
                             scale=1.0 / math.sqrt(DPH))
    return pl.pallas_call(
        body,
        out_shape=jax.ShapeDtypeStruct((B, N, D), jnp.float32),
        grid=(B, G),
        in_specs=[
            pl.BlockSpec((1, N, D), lambda b, g: (b, 0, 0)),
            pl.BlockSpec((1, D, 3 * q_cols), lambda b, g: (g, 0, 0)),
            pl.BlockSpec((1, 1, 3 * q_cols), lambda b, g: (g, 0, 0)),
            pl.BlockSpec((1, q_cols, D), lambda b, g: (g, 0, 0)),
            pl.BlockSpec((1, D), lambda b, g: (0, 0)),
        ],
        out_specs=pl.BlockSpec((1, N, D), lambda b, g: (b, 0, 0)),
        scratch_shapes=[pltpu.VMEM((q_cols, N), jnp.bfloat16)],
        compiler_params=pltpu.CompilerParams(
            dimension_semantics=("parallel", "arbitrary"),
            vmem_limit_bytes=63 * 1024 * 1024,
        ),
        name="fused_mha",
    )(x.astype(jnp.bfloat16), Wqkv, bqkv, Wog, bo2)


# speedup vs baseline: 1.0790x; 1.0790x over previous
import functools
import math

import jax
import jax.numpy as jnp
from jax.experimental import pallas as pl
from jax.experimental.pallas import tpu as pltpu

HPG = 4
CHUNK = 512


def _attn_kernel(x_ref, wqkv_ref, bqkv_ref, wo_ref, bo_ref, out_ref, ctx_ref,
                 *, n, dph, scale):
    g = pl.program_id(1)
    q_cols = HPG * dph

    xb = x_ref[0]
    qkv = jax.lax.dot_general(
        xb, wqkv_ref[0], (((1,), (0,)), ((), ())),
        preferred_element_type=jnp.float32) + bqkv_ref[0]

    log2e_scale = scale * 1.4426950408889634
    for h in range(HPG):
        q = (qkv[:, h * dph:(h + 1) * dph] * log2e_scale).astype(jnp.bfloat16)
        k = qkv[:, q_cols + h * dph:q_cols + (h + 1) * dph].astype(
            jnp.bfloat16)
        v = qkv[:, 2 * q_cols + h * dph:2 * q_cols + (h + 1) * dph]
        for c in range(n // CHUNK):
            rows = slice(c * CHUNK, (c + 1) * CHUNK)
            s = jax.lax.dot_general(
                q[rows], k, (((1,), (1,)), ((), ())),
                preferred_element_type=jnp.float32)
            m = jnp.max(s, axis=1, keepdims=True)
            e = jnp.exp2(s - m)
            l = jnp.sum(e, axis=1, keepdims=True)
            cc = jnp.dot(e, v, preferred_element_type=jnp.float32) / l
            ctx_ref[rows, h * dph:(h + 1) * dph] = cc.astype(jnp.bfloat16)

    wo = wo_ref[0]
    for c in range(n // CHUNK):
        rows = slice(c * CHUNK, (c + 1) * CHUNK)
        contrib = jnp.dot(ctx_ref[rows, :], wo,
                          preferred_element_type=jnp.float32)

        @pl.when(g == 0)
        def _():
            out_ref[0, rows, :] = contrib + bo_ref[...]

        @pl.when(g != 0)
        def _():
            out_ref[0, rows, :] = out_ref[0, rows, :] + contrib


def kernel(x, Wq, bq, Wk, bk, Wv, bv, Wo, bo):
    B, N, D = x.shape
    H, _, DPH = Wq.shape
    G = H // HPG
    q_cols = HPG * DPH

    def group_w(W):
        return W.reshape(G, HPG, D, DPH).transpose(0, 2, 1, 3).reshape(
            G, D, q_cols)

    Wqkv = jnp.concatenate([group_w(Wq), group_w(Wk), group_w(Wv)],
                           axis=2).astype(jnp.bfloat16)
    bqkv = jnp.concatenate(
        [bq.reshape(G, 1, q_cols), bk.reshape(G, 1, q_cols),
         bv.reshape(G, 1, q_cols)], axis=2)
    Wog = Wo.reshape(G, q_cols, D).astype(jnp.bfloat16)
    bo2 = bo.reshape(1, D)

    body = functools.partial(_attn_kernel, n=N, dph=DPH,
                             scale=1.0 / math.sqrt(DPH))
    return pl.pallas_call(
        body,
        out_shape=jax.ShapeDtypeStruct((B, N, D), jnp.float32),
        grid=(B, G),
        in_specs=[
            pl.BlockSpec((1, N, D), lambda b, g: (b, 0, 0)),
            pl.BlockSpec((1, D, 3 * q_cols), lambda b, g: (g, 0, 0)),
            pl.BlockSpec((1, 1, 3 * q_cols), lambda b, g: (g, 0, 0)),
            pl.BlockSpec((1, q_cols, D), lambda b, g: (g, 0, 0)),
            pl.BlockSpec((1, D), lambda b, g: (0, 0)),
        ],
        out_specs=pl.BlockSpec((1, N, D), lambda b, g: (b, 0, 0)),
        scratch_shapes=[pltpu.VMEM((N, q_cols), jnp.bfloat16)],
        compiler_params=pltpu.CompilerParams(
            dimension_semantics=("parallel", "arbitrary"),
            vmem_limit_bytes=63 * 1024 * 1024,
        ),
        name="fused_mha",
    )(x.astype(jnp.bfloat16), Wqkv, bqkv, Wog, bo2)
